# baseline (speedup 1.0000x reference)
import jax
import jax.numpy as jnp
from jax.experimental import pallas as pl
from jax.experimental.pallas import tpu as pltpu


def _bcast_body(t_ref, emb_ref, out_ref):
    del t_ref
    out_ref[...] = jax.lax.broadcast_in_dim(emb_ref[0], out_ref.shape, (1, 2))


def kernel(x, t, embeddings):
    B, _, H, W = x.shape
    C = embeddings.shape[1]
    HW = H * W
    CB = 32
    t_arr = jnp.asarray(t, jnp.int32).reshape((1,))
    emb3 = embeddings.reshape(embeddings.shape[0], C, 1)
    grid_spec = pltpu.PrefetchScalarGridSpec(
        num_scalar_prefetch=1,
        grid=(C // CB,),
        in_specs=[pl.BlockSpec((1, CB, 1), lambda i, tr: (tr[0], i, 0))],
        out_specs=pl.BlockSpec((B, CB, HW), lambda i, tr: (0, i, 0)),
    )
    out = pl.pallas_call(
        _bcast_body,
        grid_spec=grid_spec,
        out_shape=jax.ShapeDtypeStruct((B, C, HW), jnp.float32),
        compiler_params=pltpu.CompilerParams(
            dimension_semantics=("parallel",),
        ),
    )(t_arr, emb3)
    return out.reshape(B, C, H, W)
